# baseline (speedup 1.0000x reference)
import functools

import jax
import jax.numpy as jnp
from jax import lax
from jax.experimental import pallas as pl
from jax.experimental.pallas import tpu as pltpu

_F = 16
_K = 20


def _frame_body(N, K, G, FD, L1, OUT,
                feats_ref, gt_ref, wpre_ref, bpre_ref, wcat_ref,
                w2_ref, b2_ref, a2_ref, ab2_ref, lng_ref, lnb_ref,
                out_ref, d2_ref):
    feats = feats_ref[...]
    f32 = jnp.float32

    def build_d2():
        d2 = jnp.zeros((N, N), f32)
        for c in range(G):
            col = feats[:, c:c + 1]
            row = gt_ref[c:c + 1, :]
            dif = col - row
            d2 = d2 + dif * dif
        rows = lax.broadcasted_iota(jnp.int32, (N, N), 0)
        cols = lax.broadcasted_iota(jnp.int32, (N, N), 1)
        return jnp.where(rows == cols, d2 + 1e10, d2)

    d2_ref[...] = build_d2()

    pre = jnp.dot(feats, wpre_ref[...],
                  preferred_element_type=f32) + bpre_ref[...]

    def mlp(nbr, acc):
        t = jax.nn.relu(pre + jnp.dot(nbr, wcat_ref[...],
                                      preferred_element_type=f32))
        h = t[:, :L1]
        ha = t[:, L1:]
        ek = jax.nn.relu(jnp.dot(h, w2_ref[...], preferred_element_type=f32)
                         + b2_ref[...])
        gb = jnp.dot(ha, a2_ref[...], preferred_element_type=f32) + ab2_ref[...]
        gam = jax.nn.sigmoid(gb[:, :OUT] + 1.0)
        bet = gb[:, OUT:]
        return jnp.maximum(acc, gam * ek + bet)

    def fast_step(_, carry):
        acc, cnt, lastv = carry
        dv = d2_ref[...]
        cand = jnp.where(dv > lastv, dv, 3e38)
        rmin = jnp.min(cand, axis=1, keepdims=True)
        m = cand == rmin
        nbr = jnp.dot(m.astype(f32), feats, preferred_element_type=f32)
        return mlp(nbr, acc), cnt + nbr[:, FD:FD + 1], rmin

    acc0 = jnp.full((N, OUT), -jnp.inf, f32)
    acc_fast, cnt, _ = lax.fori_loop(
        0, K, fast_step,
        (acc0, jnp.zeros((N, 1), f32), jnp.full((N, 1), -1.0, f32)))
    total = jnp.sum(cnt)

    def exact():
        cols = lax.broadcasted_iota(jnp.int32, (N, N), 1)

        def step(_, acc):
            dv = d2_ref[...]
            rmin = jnp.min(dv, axis=1, keepdims=True)
            idx = jnp.where(dv == rmin, cols, N)
            amin = jnp.min(idx, axis=1, keepdims=True)
            onehot = cols == amin
            d2_ref[...] = jnp.where(onehot, 3e38, dv)
            nbr = jnp.dot(onehot.astype(f32), feats,
                          preferred_element_type=f32)
            return mlp(nbr, acc)

        return lax.fori_loop(0, K, step, acc0)

    acc = lax.cond(total == float(N * K), lambda: acc_fast, exact)

    mu = jnp.mean(acc, axis=1, keepdims=True)
    xc = acc - mu
    var = jnp.mean(xc * xc, axis=1, keepdims=True)
    y = xc * lax.rsqrt(var + 1e-5) * lng_ref[...] + lnb_ref[...]
    out_ref[...] = jax.nn.relu(y)


def _edgeconv(geom, aux, W1, b1, W2, b2, A1, ab1, A2, ab2, ln_g, ln_b,
              frames, k):
    P, G = geom.shape
    A = aux.shape[1]
    N = P // frames
    FD = G + A
    L1 = W1.shape[1]
    HA = A1.shape[1]
    OUT = W2.shape[1]
    TW = L1 + HA

    f32 = jnp.float32
    feats = jnp.concatenate(
        [geom, aux, jnp.ones((P, 1), f32)], axis=1)
    geomT = geom.T
    wpre = jnp.zeros((FD + 1, TW), f32)
    wpre = wpre.at[:G, :L1].set(W1[:G] - W1[G:])
    wpre = wpre.at[G:FD, L1:].set(A1[:A])
    wcat = jnp.zeros((FD + 1, TW), f32)
    wcat = wcat.at[:G, :L1].set(W1[G:])
    wcat = wcat.at[G:FD, L1:].set(A1[A:])
    bpre = jnp.concatenate([b1, ab1]).reshape(1, TW)

    body = functools.partial(_frame_body, N, k, G, FD, L1, OUT)
    full = lambda i: (0, 0)
    out = pl.pallas_call(
        body,
        grid=(frames,),
        in_specs=[
            pl.BlockSpec((N, FD + 1), lambda i: (i, 0)),
            pl.BlockSpec((G, N), lambda i: (0, i)),
            pl.BlockSpec((FD + 1, TW), full),
            pl.BlockSpec((1, TW), full),
            pl.BlockSpec((FD + 1, TW), full),
            pl.BlockSpec((L1, OUT), full),
            pl.BlockSpec((1, OUT), full),
            pl.BlockSpec((HA, 2 * OUT), full),
            pl.BlockSpec((1, 2 * OUT), full),
            pl.BlockSpec((1, OUT), full),
            pl.BlockSpec((1, OUT), full),
        ],
        out_specs=pl.BlockSpec((N, OUT), lambda i: (i, 0)),
        out_shape=jax.ShapeDtypeStruct((P, OUT), f32),
        scratch_shapes=[pltpu.VMEM((N, N), f32)],
    )(feats, geomT, wpre, bpre, wcat, W2, b2.reshape(1, OUT), A2,
      ab2.reshape(1, 2 * OUT), ln_g.reshape(1, OUT), ln_b.reshape(1, OUT))
    return out


def kernel(geom, aux, batch, W1, b1, W2, b2, A1, ab1, A2, ab2, ln_g, ln_b):
    del batch
    return _edgeconv(geom, aux, W1, b1, W2, b2, A1, ab1, A2, ab2,
                     ln_g, ln_b, _F, _K)


# speedup vs baseline: 1.1798x; 1.1798x over previous
import functools

import jax
import jax.numpy as jnp
from jax import lax
from jax.experimental import pallas as pl
from jax.experimental.pallas import tpu as pltpu

_F = 16
_K = 20


def _frame_body(N, K, G, FD, L1, OUT,
                feats_ref, gt_ref, wpre_ref, bpre_ref, wcat_ref,
                w2_ref, b2_ref, a2_ref, ab2_ref, lng_ref, lnb_ref,
                out_ref, d2_ref):
    feats = feats_ref[...]
    f32 = jnp.float32

    def build_d2():
        d2 = jnp.zeros((N, N), f32)
        for c in range(G):
            col = feats[:, c:c + 1]
            row = gt_ref[c:c + 1, :]
            dif = col - row
            d2 = d2 + dif * dif
        rows = lax.broadcasted_iota(jnp.int32, (N, N), 0)
        cols = lax.broadcasted_iota(jnp.int32, (N, N), 1)
        return jnp.where(rows == cols, d2 + 1e10, d2)

    d2_ref[...] = build_d2()

    pre = jnp.dot(feats, wpre_ref[...],
                  preferred_element_type=f32) + bpre_ref[...]

    def mlp(nbr, acc):
        t = jax.nn.relu(pre + jnp.dot(nbr, wcat_ref[...],
                                      preferred_element_type=f32))
        h = t[:, :L1]
        ha = t[:, L1:]
        ek = jax.nn.relu(jnp.dot(h, w2_ref[...], preferred_element_type=f32)
                         + b2_ref[...])
        gb = jnp.dot(ha, a2_ref[...], preferred_element_type=f32) + ab2_ref[...]
        gam = jax.nn.sigmoid(gb[:, :OUT] + 1.0)
        bet = gb[:, OUT:]
        return jnp.maximum(acc, gam * ek + bet)

    def select(cnt):
        dv = d2_ref[...]
        rmin = jnp.min(dv, axis=1, keepdims=True)
        m = dv == rmin
        d2_ref[...] = jnp.where(m, 3e38, dv)
        nbr = jnp.dot(m.astype(f32), feats, preferred_element_type=f32)
        return nbr, cnt + nbr[:, FD:FD + 1]

    acc0 = jnp.full((N, OUT), -jnp.inf, f32)
    nbr_p, cnt0 = select(jnp.zeros((N, 1), f32))

    def fast_step(_, carry):
        acc, cnt, nbr_prev = carry
        nbr, cnt = select(cnt)
        return mlp(nbr_prev, acc), cnt, nbr

    acc_fast, cnt, nbr_p = lax.fori_loop(
        1, K, fast_step, (acc0, cnt0, nbr_p))
    acc_fast = mlp(nbr_p, acc_fast)
    total = jnp.sum(cnt)

    def exact():
        d2_ref[...] = build_d2()
        cols = lax.broadcasted_iota(jnp.int32, (N, N), 1)

        def step(_, acc):
            dv = d2_ref[...]
            rmin = jnp.min(dv, axis=1, keepdims=True)
            idx = jnp.where(dv == rmin, cols, N)
            amin = jnp.min(idx, axis=1, keepdims=True)
            onehot = cols == amin
            d2_ref[...] = jnp.where(onehot, 3e38, dv)
            nbr = jnp.dot(onehot.astype(f32), feats,
                          preferred_element_type=f32)
            return mlp(nbr, acc)

        return lax.fori_loop(0, K, step, acc0)

    acc = lax.cond(total == float(N * K), lambda: acc_fast, exact)

    mu = jnp.mean(acc, axis=1, keepdims=True)
    xc = acc - mu
    var = jnp.mean(xc * xc, axis=1, keepdims=True)
    y = xc * lax.rsqrt(var + 1e-5) * lng_ref[...] + lnb_ref[...]
    out_ref[...] = jax.nn.relu(y)


def _edgeconv(geom, aux, W1, b1, W2, b2, A1, ab1, A2, ab2, ln_g, ln_b,
              frames, k):
    P, G = geom.shape
    A = aux.shape[1]
    N = P // frames
    FD = G + A
    L1 = W1.shape[1]
    HA = A1.shape[1]
    OUT = W2.shape[1]
    TW = L1 + HA

    f32 = jnp.float32
    feats = jnp.concatenate(
        [geom, aux, jnp.ones((P, 1), f32)], axis=1)
    geomT = geom.T
    wpre = jnp.zeros((FD + 1, TW), f32)
    wpre = wpre.at[:G, :L1].set(W1[:G] - W1[G:])
    wpre = wpre.at[G:FD, L1:].set(A1[:A])
    wcat = jnp.zeros((FD + 1, TW), f32)
    wcat = wcat.at[:G, :L1].set(W1[G:])
    wcat = wcat.at[G:FD, L1:].set(A1[A:])
    bpre = jnp.concatenate([b1, ab1]).reshape(1, TW)

    body = functools.partial(_frame_body, N, k, G, FD, L1, OUT)
    full = lambda i: (0, 0)
    out = pl.pallas_call(
        body,
        grid=(frames,),
        in_specs=[
            pl.BlockSpec((N, FD + 1), lambda i: (i, 0)),
            pl.BlockSpec((G, N), lambda i: (0, i)),
            pl.BlockSpec((FD + 1, TW), full),
            pl.BlockSpec((1, TW), full),
            pl.BlockSpec((FD + 1, TW), full),
            pl.BlockSpec((L1, OUT), full),
            pl.BlockSpec((1, OUT), full),
            pl.BlockSpec((HA, 2 * OUT), full),
            pl.BlockSpec((1, 2 * OUT), full),
            pl.BlockSpec((1, OUT), full),
            pl.BlockSpec((1, OUT), full),
        ],
        out_specs=pl.BlockSpec((N, OUT), lambda i: (i, 0)),
        out_shape=jax.ShapeDtypeStruct((P, OUT), f32),
        scratch_shapes=[pltpu.VMEM((N, N), f32)],
    )(feats, geomT, wpre, bpre, wcat, W2, b2.reshape(1, OUT), A2,
      ab2.reshape(1, 2 * OUT), ln_g.reshape(1, OUT), ln_b.reshape(1, OUT))
    return out


def kernel(geom, aux, batch, W1, b1, W2, b2, A1, ab1, A2, ab2, ln_g, ln_b):
    del batch
    return _edgeconv(geom, aux, W1, b1, W2, b2, A1, ab1, A2, ab2,
                     ln_g, ln_b, _F, _K)
